# baseline (speedup 1.0000x reference)
import functools

import jax
import jax.numpy as jnp
from jax import lax
from jax.experimental import pallas as pl
from jax.experimental.pallas import tpu as pltpu
from jax.experimental.pallas import tpu_sc as plsc

N_NODES = 10000
N_EDGES = 320000
D_EDGE = 16
D_FEAT = 128
D_GLOBAL = 128

NUM_CORES = 2
NUM_SUBCORES = 16
NW = NUM_CORES * NUM_SUBCORES
G = 128
NGROUPS = N_EDGES // G
GPW = NGROUPS // NW
EXTRA = NGROUPS - GPW * NW
KG = 13
MEGA = GPW // KG
CHUNK = KG * G
EPW = GPW * G
STRIPE = 624
TAIL = N_NODES - NUM_SUBCORES * STRIPE


def _sc_segment_sum(dst, ea3):
  mesh = plsc.VectorSubcoreMesh(core_axis_name="c", subcore_axis_name="s")

  @functools.partial(
      pl.kernel,
      out_type=(
          jax.ShapeDtypeStruct((NUM_CORES, N_NODES, D_EDGE), jnp.float32),
          jax.ShapeDtypeStruct((NUM_CORES, N_NODES, D_EDGE), jnp.float32),
      ),
      mesh=mesh,
      compiler_params=pltpu.CompilerParams(use_tc_tiling_on_sc=False, needs_layout_passes=False),
      scratch_types=[
          pltpu.VMEM((EPW + EXTRA * G,), jnp.int32),
          pltpu.VMEM((KG, D_EDGE, G), jnp.float32),
          pltpu.VMEM((KG, D_EDGE, G), jnp.float32),
          pltpu.VMEM((CHUNK, D_EDGE), jnp.float32),
          pltpu.VMEM((G, D_EDGE), jnp.float32),
          pltpu.VMEM_SHARED((N_NODES, D_EDGE), jnp.float32),
          pltpu.VMEM_SHARED((N_NODES, D_EDGE), jnp.float32),
          pltpu.SemaphoreType.DMA,
          pltpu.SemaphoreType.DMA,
          pltpu.SemaphoreType.DMA,
      ],
  )
  def k(dst_hbm, ea_hbm, sums_hbm, counts_hbm, idx_v, fb_a, fb_b, rows_v,
        ones_v, sum_s, cnt_s, sem_i, sem_a, sem_b):
    c = lax.axis_index("c")
    s = lax.axis_index("s")
    w = c * NUM_SUBCORES + s
    g0 = w * GPW

    idx_dma = pltpu.async_copy(dst_hbm.at[1, pl.ds(w * EPW, EPW)],
                               idx_v.at[pl.ds(0, EPW)], sem_i)

    @pl.when(w < EXTRA)
    def _():
      pltpu.async_copy(dst_hbm.at[1, pl.ds(NW * EPW + w * G, G)],
                       idx_v.at[pl.ds(EPW, G)], sem_i)

    @pl.loop(0, STRIPE)
    def _(i):
      rows_v[i, :] = jnp.zeros((16,), jnp.float32)

    @pl.loop(0, G)
    def _(i):
      ones_v[i, :] = jnp.ones((16,), jnp.float32)

    base_row = s * STRIPE
    pltpu.sync_copy(rows_v.at[pl.ds(0, STRIPE)],
                    sum_s.at[pl.ds(base_row, STRIPE)])
    pltpu.sync_copy(rows_v.at[pl.ds(0, STRIPE)],
                    cnt_s.at[pl.ds(base_row, STRIPE)])

    @pl.when(s == 0)
    def _():
      pltpu.sync_copy(rows_v.at[pl.ds(0, TAIL)],
                      sum_s.at[pl.ds(NUM_SUBCORES * STRIPE, TAIL)])
      pltpu.sync_copy(rows_v.at[pl.ds(0, TAIL)],
                      cnt_s.at[pl.ds(NUM_SUBCORES * STRIPE, TAIL)])

    idx_dma.wait()

    @pl.when(w < EXTRA)
    def _():
      pltpu.make_async_copy(dst_hbm.at[1, pl.ds(NW * EPW + w * G, G)],
                            idx_v.at[pl.ds(EPW, G)], sem_i).wait()

    plsc.subcore_barrier()

    iota16 = lax.iota(jnp.int32, 16)

    def transpose_and_scatter(fbuf, ngroups, idx_base):
      zeros16 = iota16 * 0

      @pl.loop(0, ngroups)
      def _(g):
        g_vec = zeros16 + g

        @pl.loop(0, G // 16)
        def _(l16):
          l_vec = zeros16 + l16 * 16
          e_base = g * G + l16 * 16
          for dl in range(16):
            vals = plsc.load_gather(fbuf, [g_vec, iota16, l_vec + dl])
            rows_v[e_base + dl, :] = vals

        idx = idx_v.at[pl.ds(idx_base + g * G, G)]
        pltpu.sync_copy(rows_v.at[pl.ds(g * G, G)], sum_s.at[idx], add=True)
        pltpu.sync_copy(ones_v, cnt_s.at[idx], add=True)

    bufs = (fb_a, fb_b)
    sems = (sem_a, sem_b)
    chunk_dmas = [None] * (MEGA + 1)

    def start(m):
      if m < MEGA:
        src = ea_hbm.at[pl.ds(g0 + m * KG, KG)]
        dst_buf = bufs[m % 2]
      else:
        src = ea_hbm.at[pl.ds(NW * GPW + w, 1)]
        dst_buf = bufs[m % 2].at[pl.ds(0, 1)]
      chunk_dmas[m] = pltpu.async_copy(src, dst_buf, sems[m % 2])

    start(0)
    for m in range(MEGA):
      if m + 1 < MEGA:
        start(m + 1)
      elif EXTRA:
        @pl.when(w < EXTRA)
        def _(m=m):
          start(m + 1)
      chunk_dmas[m].wait()
      transpose_and_scatter(bufs[m % 2], KG, m * CHUNK)

    if EXTRA:
      @pl.when(w < EXTRA)
      def _():
        pltpu.make_async_copy(ea_hbm.at[pl.ds(NW * GPW + w, 1)],
                              bufs[MEGA % 2].at[pl.ds(0, 1)],
                              sems[MEGA % 2]).wait()
        transpose_and_scatter(bufs[MEGA % 2], 1, EPW)

    plsc.subcore_barrier()

    pltpu.sync_copy(sum_s.at[pl.ds(base_row, STRIPE)],
                    sums_hbm.at[c, pl.ds(base_row, STRIPE)])
    pltpu.sync_copy(cnt_s.at[pl.ds(base_row, STRIPE)],
                    counts_hbm.at[c, pl.ds(base_row, STRIPE)])

    @pl.when(s == 0)
    def _():
      pltpu.sync_copy(sum_s.at[pl.ds(NUM_SUBCORES * STRIPE, TAIL)],
                      sums_hbm.at[c, pl.ds(NUM_SUBCORES * STRIPE, TAIL)])
      pltpu.sync_copy(cnt_s.at[pl.ds(NUM_SUBCORES * STRIPE, TAIL)],
                      counts_hbm.at[c, pl.ds(NUM_SUBCORES * STRIPE, TAIL)])

  return k(dst, ea3)


def _tc_finish_body(x_ref, u_ref, s_ref, c_ref, o_ref):
  total = s_ref[0] + s_ref[1]
  cnt = c_ref[0] + c_ref[1]
  agg = total / jnp.maximum(cnt, 1.0)
  u_b = jnp.broadcast_to(u_ref[...], (N_NODES, D_GLOBAL))
  o_ref[...] = jnp.concatenate([agg, x_ref[...], u_b], axis=1)


def _tc_finish(x, u2d, sums, counts):
  return pl.pallas_call(
      _tc_finish_body,
      out_shape=jax.ShapeDtypeStruct(
          (N_NODES, D_EDGE + D_FEAT + D_GLOBAL), jnp.float32),
  )(x, u2d, sums, counts)


@jax.jit
def kernel(x, edge_attr, u, edge_index):
  dst = edge_index.astype(jnp.int32)
  ea3 = edge_attr.T.reshape(D_EDGE, NGROUPS, G).transpose(1, 0, 2)
  sums, counts = _sc_segment_sum(dst, ea3)
  return _tc_finish(x, u.reshape(1, D_GLOBAL), sums, counts)


# speedup vs baseline: 1.3885x; 1.3885x over previous
import functools

import jax
import jax.numpy as jnp
from jax import lax
from jax.experimental import pallas as pl
from jax.experimental.pallas import tpu as pltpu
from jax.experimental.pallas import tpu_sc as plsc

N_NODES = 10000
N_EDGES = 320000
D_EDGE = 16
D_FEAT = 128
D_GLOBAL = 128

NUM_CORES = 2
NUM_SUBCORES = 16
NW = NUM_CORES * NUM_SUBCORES
G = 128
NGROUPS = N_EDGES // G
GPW = NGROUPS // NW
EXTRA = NGROUPS - GPW * NW
KG = 13
MEGA = GPW // KG
CHUNK = KG * G
EPW = GPW * G
STRIPE = 624
TAIL = N_NODES - NUM_SUBCORES * STRIPE


def _sc_segment_sum(dst, ea3):
  mesh = plsc.VectorSubcoreMesh(core_axis_name="c", subcore_axis_name="s")

  @functools.partial(
      pl.kernel,
      out_type=(
          jax.ShapeDtypeStruct((NUM_CORES, N_NODES, D_EDGE), jnp.float32),
          jax.ShapeDtypeStruct((NUM_CORES, N_NODES, D_EDGE), jnp.float32),
      ),
      mesh=mesh,
      compiler_params=pltpu.CompilerParams(use_tc_tiling_on_sc=False, needs_layout_passes=False),
      scratch_types=[
          pltpu.VMEM((EPW + EXTRA * G,), jnp.int32),
          pltpu.VMEM((KG, D_EDGE, G + 1), jnp.float32),
          pltpu.VMEM((KG, D_EDGE, G + 1), jnp.float32),
          pltpu.VMEM((CHUNK, D_EDGE), jnp.float32),
          pltpu.VMEM((G, D_EDGE), jnp.float32),
          pltpu.VMEM_SHARED((N_NODES, D_EDGE), jnp.float32),
          pltpu.VMEM_SHARED((N_NODES, D_EDGE), jnp.float32),
          pltpu.SemaphoreType.DMA,
          pltpu.SemaphoreType.DMA,
          pltpu.SemaphoreType.DMA,
      ],
  )
  def k(dst_hbm, ea_hbm, sums_hbm, counts_hbm, idx_v, fb_a, fb_b, rows_v,
        ones_v, sum_s, cnt_s, sem_i, sem_a, sem_b):
    c = lax.axis_index("c")
    s = lax.axis_index("s")
    w = c * NUM_SUBCORES + s
    g0 = w * GPW

    idx_dma = pltpu.async_copy(dst_hbm.at[1, pl.ds(w * EPW, EPW)],
                               idx_v.at[pl.ds(0, EPW)], sem_i)

    @pl.when(w < EXTRA)
    def _():
      pltpu.async_copy(dst_hbm.at[1, pl.ds(NW * EPW + w * G, G)],
                       idx_v.at[pl.ds(EPW, G)], sem_i)

    @pl.loop(0, STRIPE)
    def _(i):
      rows_v[i, :] = jnp.zeros((16,), jnp.float32)

    @pl.loop(0, G)
    def _(i):
      ones_v[i, :] = jnp.ones((16,), jnp.float32)

    base_row = s * STRIPE
    pltpu.sync_copy(rows_v.at[pl.ds(0, STRIPE)],
                    sum_s.at[pl.ds(base_row, STRIPE)])
    pltpu.sync_copy(rows_v.at[pl.ds(0, STRIPE)],
                    cnt_s.at[pl.ds(base_row, STRIPE)])

    @pl.when(s == 0)
    def _():
      pltpu.sync_copy(rows_v.at[pl.ds(0, TAIL)],
                      sum_s.at[pl.ds(NUM_SUBCORES * STRIPE, TAIL)])
      pltpu.sync_copy(rows_v.at[pl.ds(0, TAIL)],
                      cnt_s.at[pl.ds(NUM_SUBCORES * STRIPE, TAIL)])

    idx_dma.wait()

    @pl.when(w < EXTRA)
    def _():
      pltpu.make_async_copy(dst_hbm.at[1, pl.ds(NW * EPW + w * G, G)],
                            idx_v.at[pl.ds(EPW, G)], sem_i).wait()

    plsc.subcore_barrier()

    iota16 = lax.iota(jnp.int32, 16)

    def transpose_and_scatter(fbuf, ngroups, idx_base):
      zeros16 = iota16 * 0

      @pl.loop(0, ngroups)
      def _(g):
        g_vec = zeros16 + g

        @pl.loop(0, G // 16)
        def _(l16):
          l_vec = zeros16 + l16 * 16
          e_base = g * G + l16 * 16
          for dl in range(16):
            vals = plsc.load_gather(fbuf, [g_vec, iota16, l_vec + dl])
            rows_v[e_base + dl, :] = vals

        idx = idx_v.at[pl.ds(idx_base + g * G, G)]
        pltpu.sync_copy(rows_v.at[pl.ds(g * G, G)], sum_s.at[idx], add=True)
        pltpu.sync_copy(ones_v, cnt_s.at[idx], add=True)

    bufs = (fb_a, fb_b)
    sems = (sem_a, sem_b)
    chunk_dmas = [None] * (MEGA + 1)

    def start(m):
      if m < MEGA:
        src = ea_hbm.at[pl.ds(g0 + m * KG, KG)]
        dst_buf = bufs[m % 2].at[:, :, pl.ds(0, G)]
      else:
        src = ea_hbm.at[pl.ds(NW * GPW + w, 1)]
        dst_buf = bufs[m % 2].at[pl.ds(0, 1), :, pl.ds(0, G)]
      chunk_dmas[m] = pltpu.async_copy(src, dst_buf, sems[m % 2])

    start(0)
    for m in range(MEGA):
      if m + 1 < MEGA:
        start(m + 1)
      elif EXTRA:
        @pl.when(w < EXTRA)
        def _(m=m):
          start(m + 1)
      chunk_dmas[m].wait()
      transpose_and_scatter(bufs[m % 2], KG, m * CHUNK)

    if EXTRA:
      @pl.when(w < EXTRA)
      def _():
        pltpu.make_async_copy(ea_hbm.at[pl.ds(NW * GPW + w, 1)],
                              bufs[MEGA % 2].at[pl.ds(0, 1), :, pl.ds(0, G)],
                              sems[MEGA % 2]).wait()
        transpose_and_scatter(bufs[MEGA % 2], 1, EPW)

    plsc.subcore_barrier()

    pltpu.sync_copy(sum_s.at[pl.ds(base_row, STRIPE)],
                    sums_hbm.at[c, pl.ds(base_row, STRIPE)])
    pltpu.sync_copy(cnt_s.at[pl.ds(base_row, STRIPE)],
                    counts_hbm.at[c, pl.ds(base_row, STRIPE)])

    @pl.when(s == 0)
    def _():
      pltpu.sync_copy(sum_s.at[pl.ds(NUM_SUBCORES * STRIPE, TAIL)],
                      sums_hbm.at[c, pl.ds(NUM_SUBCORES * STRIPE, TAIL)])
      pltpu.sync_copy(cnt_s.at[pl.ds(NUM_SUBCORES * STRIPE, TAIL)],
                      counts_hbm.at[c, pl.ds(NUM_SUBCORES * STRIPE, TAIL)])

  return k(dst, ea3)


def _tc_finish_body(x_ref, u_ref, s_ref, c_ref, o_ref):
  total = s_ref[0] + s_ref[1]
  cnt = c_ref[0] + c_ref[1]
  agg = total / jnp.maximum(cnt, 1.0)
  u_b = jnp.broadcast_to(u_ref[...], (N_NODES, D_GLOBAL))
  o_ref[...] = jnp.concatenate([agg, x_ref[...], u_b], axis=1)


def _tc_finish(x, u2d, sums, counts):
  return pl.pallas_call(
      _tc_finish_body,
      out_shape=jax.ShapeDtypeStruct(
          (N_NODES, D_EDGE + D_FEAT + D_GLOBAL), jnp.float32),
  )(x, u2d, sums, counts)


@jax.jit
def kernel(x, edge_attr, u, edge_index):
  dst = edge_index.astype(jnp.int32)
  ea3 = edge_attr.T.reshape(D_EDGE, NGROUPS, G).transpose(1, 0, 2)
  sums, counts = _sc_segment_sum(dst, ea3)
  return _tc_finish(x, u.reshape(1, D_GLOBAL), sums, counts)


# speedup vs baseline: 1.7437x; 1.2558x over previous
import functools

import jax
import jax.numpy as jnp
from jax import lax
from jax.experimental import pallas as pl
from jax.experimental.pallas import tpu as pltpu
from jax.experimental.pallas import tpu_sc as plsc

N_NODES = 10000
N_EDGES = 320000
D_EDGE = 16
D_FEAT = 128
D_GLOBAL = 128

NUM_CORES = 2
NUM_SUBCORES = 16
NW = NUM_CORES * NUM_SUBCORES
G = 128
NGROUPS = N_EDGES // G
GPW = NGROUPS // NW
EXTRA = NGROUPS - GPW * NW
KG = 13
MEGA = GPW // KG
CHUNK = KG * G
EPW = GPW * G
STRIPE = 624
TAIL = N_NODES - NUM_SUBCORES * STRIPE


def _sc_segment_sum(dst, ea3):
  mesh = plsc.VectorSubcoreMesh(core_axis_name="c", subcore_axis_name="s")

  @functools.partial(
      pl.kernel,
      out_type=(
          jax.ShapeDtypeStruct((NUM_CORES, N_NODES, D_EDGE), jnp.float32),
          jax.ShapeDtypeStruct((NUM_CORES, N_NODES, D_EDGE), jnp.float32),
      ),
      mesh=mesh,
      compiler_params=pltpu.CompilerParams(use_tc_tiling_on_sc=False, needs_layout_passes=False),
      scratch_types=[
          pltpu.VMEM((EPW + EXTRA * G,), jnp.int32),
          pltpu.VMEM((KG, D_EDGE, G + 1), jnp.float32),
          pltpu.VMEM((KG, D_EDGE, G + 1), jnp.float32),
          pltpu.VMEM((CHUNK, D_EDGE), jnp.float32),
          pltpu.VMEM((G, D_EDGE), jnp.float32),
          pltpu.VMEM_SHARED((N_NODES, D_EDGE), jnp.float32),
          pltpu.VMEM_SHARED((N_NODES, D_EDGE), jnp.float32),
          pltpu.SemaphoreType.DMA,
          pltpu.SemaphoreType.DMA,
          pltpu.SemaphoreType.DMA,
      ],
  )
  def k(dst_hbm, ea_hbm, sums_hbm, counts_hbm, idx_v, fb_a, fb_b, rows_v,
        ones_v, sum_s, cnt_s, sem_i, sem_a, sem_b):
    c = lax.axis_index("c")
    s = lax.axis_index("s")
    w = c * NUM_SUBCORES + s
    g0 = w * GPW

    idx_dma = pltpu.async_copy(dst_hbm.at[1, pl.ds(w * EPW, EPW)],
                               idx_v.at[pl.ds(0, EPW)], sem_i)

    @pl.when(w < EXTRA)
    def _():
      pltpu.async_copy(dst_hbm.at[1, pl.ds(NW * EPW + w * G, G)],
                       idx_v.at[pl.ds(EPW, G)], sem_i)

    @pl.loop(0, STRIPE)
    def _(i):
      rows_v[i, :] = jnp.zeros((16,), jnp.float32)

    @pl.loop(0, G)
    def _(i):
      ones_v[i, :] = jnp.ones((16,), jnp.float32)

    base_row = s * STRIPE
    pltpu.sync_copy(rows_v.at[pl.ds(0, STRIPE)],
                    sum_s.at[pl.ds(base_row, STRIPE)])
    pltpu.sync_copy(rows_v.at[pl.ds(0, STRIPE)],
                    cnt_s.at[pl.ds(base_row, STRIPE)])

    @pl.when(s == 0)
    def _():
      pltpu.sync_copy(rows_v.at[pl.ds(0, TAIL)],
                      sum_s.at[pl.ds(NUM_SUBCORES * STRIPE, TAIL)])
      pltpu.sync_copy(rows_v.at[pl.ds(0, TAIL)],
                      cnt_s.at[pl.ds(NUM_SUBCORES * STRIPE, TAIL)])

    idx_dma.wait()

    @pl.when(w < EXTRA)
    def _():
      pltpu.make_async_copy(dst_hbm.at[1, pl.ds(NW * EPW + w * G, G)],
                            idx_v.at[pl.ds(EPW, G)], sem_i).wait()

    plsc.subcore_barrier()

    iota16 = lax.iota(jnp.int32, 16)

    def transpose_and_scatter(fbuf, ngroups, idx_base):
      zeros16 = iota16 * 0

      @pl.loop(0, ngroups)
      def _(g):
        g_vec = zeros16 + g

        @plsc.parallel_loop(0, G // 16, unroll=2)
        def _(l16):
          l_vec = zeros16 + l16 * 16
          e_base = g * G + l16 * 16
          for dl in range(16):
            vals = plsc.load_gather(fbuf, [g_vec, iota16, l_vec + dl])
            rows_v[e_base + dl, :] = vals

        idx = idx_v.at[pl.ds(idx_base + g * G, G)]
        pltpu.sync_copy(rows_v.at[pl.ds(g * G, G)], sum_s.at[idx], add=True)
        pltpu.sync_copy(ones_v, cnt_s.at[idx], add=True)

    bufs = (fb_a, fb_b)
    sems = (sem_a, sem_b)
    chunk_dmas = [None] * (MEGA + 1)

    def start(m):
      if m < MEGA:
        src = ea_hbm.at[pl.ds(g0 + m * KG, KG)]
        dst_buf = bufs[m % 2].at[:, :, pl.ds(0, G)]
      else:
        src = ea_hbm.at[pl.ds(NW * GPW + w, 1)]
        dst_buf = bufs[m % 2].at[pl.ds(0, 1), :, pl.ds(0, G)]
      chunk_dmas[m] = pltpu.async_copy(src, dst_buf, sems[m % 2])

    start(0)
    for m in range(MEGA):
      if m + 1 < MEGA:
        start(m + 1)
      elif EXTRA:
        @pl.when(w < EXTRA)
        def _(m=m):
          start(m + 1)
      chunk_dmas[m].wait()
      transpose_and_scatter(bufs[m % 2], KG, m * CHUNK)

    if EXTRA:
      @pl.when(w < EXTRA)
      def _():
        pltpu.make_async_copy(ea_hbm.at[pl.ds(NW * GPW + w, 1)],
                              bufs[MEGA % 2].at[pl.ds(0, 1), :, pl.ds(0, G)],
                              sems[MEGA % 2]).wait()
        transpose_and_scatter(bufs[MEGA % 2], 1, EPW)

    plsc.subcore_barrier()

    pltpu.sync_copy(sum_s.at[pl.ds(base_row, STRIPE)],
                    sums_hbm.at[c, pl.ds(base_row, STRIPE)])
    pltpu.sync_copy(cnt_s.at[pl.ds(base_row, STRIPE)],
                    counts_hbm.at[c, pl.ds(base_row, STRIPE)])

    @pl.when(s == 0)
    def _():
      pltpu.sync_copy(sum_s.at[pl.ds(NUM_SUBCORES * STRIPE, TAIL)],
                      sums_hbm.at[c, pl.ds(NUM_SUBCORES * STRIPE, TAIL)])
      pltpu.sync_copy(cnt_s.at[pl.ds(NUM_SUBCORES * STRIPE, TAIL)],
                      counts_hbm.at[c, pl.ds(NUM_SUBCORES * STRIPE, TAIL)])

  return k(dst, ea3)


def _tc_finish_body(x_ref, u_ref, s_ref, c_ref, o_ref):
  total = s_ref[0] + s_ref[1]
  cnt = c_ref[0] + c_ref[1]
  agg = total / jnp.maximum(cnt, 1.0)
  u_b = jnp.broadcast_to(u_ref[...], (N_NODES, D_GLOBAL))
  o_ref[...] = jnp.concatenate([agg, x_ref[...], u_b], axis=1)


def _tc_finish(x, u2d, sums, counts):
  return pl.pallas_call(
      _tc_finish_body,
      out_shape=jax.ShapeDtypeStruct(
          (N_NODES, D_EDGE + D_FEAT + D_GLOBAL), jnp.float32),
  )(x, u2d, sums, counts)


@jax.jit
def kernel(x, edge_attr, u, edge_index):
  dst = edge_index.astype(jnp.int32)
  ea3 = edge_attr.T.reshape(D_EDGE, NGROUPS, G).transpose(1, 0, 2)
  sums, counts = _sc_segment_sum(dst, ea3)
  return _tc_finish(x, u.reshape(1, D_GLOBAL), sums, counts)


# speedup vs baseline: 1.9459x; 1.1159x over previous
import functools

import jax
import jax.numpy as jnp
from jax import lax
from jax.experimental import pallas as pl
from jax.experimental.pallas import tpu as pltpu
from jax.experimental.pallas import tpu_sc as plsc

N_NODES = 10000
N_EDGES = 320000
D_EDGE = 16
D_FEAT = 128
D_GLOBAL = 128

NUM_CORES = 2
NUM_SUBCORES = 16
NW = NUM_CORES * NUM_SUBCORES
G = 128
NGROUPS = N_EDGES // G
GPW = NGROUPS // NW
EXTRA = NGROUPS - GPW * NW
KG = 13
MEGA = GPW // KG
CHUNK = KG * G
EPW = GPW * G
STRIPE = 624
TAIL = N_NODES - NUM_SUBCORES * STRIPE


def _sc_segment_sum(dst, ea3):
  mesh = plsc.VectorSubcoreMesh(core_axis_name="c", subcore_axis_name="s")

  @functools.partial(
      pl.kernel,
      out_type=(
          jax.ShapeDtypeStruct((NUM_CORES, N_NODES, D_EDGE), jnp.float32),
          jax.ShapeDtypeStruct((NUM_CORES, N_NODES, D_EDGE), jnp.float32),
      ),
      mesh=mesh,
      compiler_params=pltpu.CompilerParams(use_tc_tiling_on_sc=False, needs_layout_passes=False),
      scratch_types=[
          pltpu.VMEM((EPW + EXTRA * G,), jnp.int32),
          pltpu.VMEM((KG, D_EDGE, G + 1), jnp.float32),
          pltpu.VMEM((KG, D_EDGE, G + 1), jnp.float32),
          pltpu.VMEM((CHUNK, D_EDGE), jnp.float32),
          pltpu.VMEM((G, D_EDGE), jnp.float32),
          pltpu.VMEM_SHARED((N_NODES, D_EDGE), jnp.float32),
          pltpu.VMEM_SHARED((N_NODES, D_EDGE), jnp.float32),
          pltpu.SemaphoreType.DMA,
          pltpu.SemaphoreType.DMA,
          pltpu.SemaphoreType.DMA,
      ],
  )
  def k(dst_hbm, ea_hbm, sums_hbm, counts_hbm, idx_v, fb_a, fb_b, rows_v,
        ones_v, sum_s, cnt_s, sem_i, sem_a, sem_b):
    c = lax.axis_index("c")
    s = lax.axis_index("s")
    w = c * NUM_SUBCORES + s
    g0 = w * GPW

    idx_dma = pltpu.async_copy(dst_hbm.at[1, pl.ds(w * EPW, EPW)],
                               idx_v.at[pl.ds(0, EPW)], sem_i)

    @pl.when(w < EXTRA)
    def _():
      pltpu.async_copy(dst_hbm.at[1, pl.ds(NW * EPW + w * G, G)],
                       idx_v.at[pl.ds(EPW, G)], sem_i)

    @pl.loop(0, STRIPE)
    def _(i):
      rows_v[i, :] = jnp.zeros((16,), jnp.float32)

    @pl.loop(0, G)
    def _(i):
      ones_v[i, :] = jnp.ones((16,), jnp.float32)

    base_row = s * STRIPE
    pltpu.sync_copy(rows_v.at[pl.ds(0, STRIPE)],
                    sum_s.at[pl.ds(base_row, STRIPE)])
    pltpu.sync_copy(rows_v.at[pl.ds(0, STRIPE)],
                    cnt_s.at[pl.ds(base_row, STRIPE)])

    @pl.when(s == 0)
    def _():
      pltpu.sync_copy(rows_v.at[pl.ds(0, TAIL)],
                      sum_s.at[pl.ds(NUM_SUBCORES * STRIPE, TAIL)])
      pltpu.sync_copy(rows_v.at[pl.ds(0, TAIL)],
                      cnt_s.at[pl.ds(NUM_SUBCORES * STRIPE, TAIL)])

    idx_dma.wait()

    @pl.when(w < EXTRA)
    def _():
      pltpu.make_async_copy(dst_hbm.at[1, pl.ds(NW * EPW + w * G, G)],
                            idx_v.at[pl.ds(EPW, G)], sem_i).wait()

    plsc.subcore_barrier()

    iota16 = lax.iota(jnp.int32, 16)

    def transpose_and_scatter(fbuf, ngroups, idx_base):
      zeros16 = iota16 * 0

      @pl.loop(0, ngroups)
      def _(g):
        g_vec = zeros16 + g

        @plsc.parallel_loop(0, G // 16, unroll=2)
        def _(l16):
          l_vec = zeros16 + l16 * 16
          e_base = g * G + l16 * 16
          for dl in range(16):
            vals = plsc.load_gather(fbuf, [g_vec, iota16, l_vec + dl])
            rows_v[e_base + dl, :] = vals

        idx = idx_v.at[pl.ds(idx_base + g * G, G)]
        pltpu.sync_copy(rows_v.at[pl.ds(g * G, G)], sum_s.at[idx], add=True)
        pltpu.sync_copy(ones_v, cnt_s.at[idx], add=True)

    bufs = (fb_a, fb_b)
    sems = (sem_a, sem_b)
    chunk_dmas = [None] * (MEGA + 1)

    def start(m):
      if m < MEGA:
        src = ea_hbm.at[pl.ds(g0 + m * KG, KG)]
        dst_buf = bufs[m % 2].at[:, :, pl.ds(0, G)]
      else:
        src = ea_hbm.at[pl.ds(NW * GPW + w, 1)]
        dst_buf = bufs[m % 2].at[pl.ds(0, 1), :, pl.ds(0, G)]
      chunk_dmas[m] = pltpu.async_copy(src, dst_buf, sems[m % 2])

    start(0)
    for m in range(MEGA):
      if m + 1 < MEGA:
        start(m + 1)
      elif EXTRA:
        @pl.when(w < EXTRA)
        def _(m=m):
          start(m + 1)
      chunk_dmas[m].wait()
      transpose_and_scatter(bufs[m % 2], KG, m * CHUNK)

    if EXTRA:
      @pl.when(w < EXTRA)
      def _():
        pltpu.make_async_copy(ea_hbm.at[pl.ds(NW * GPW + w, 1)],
                              bufs[MEGA % 2].at[pl.ds(0, 1), :, pl.ds(0, G)],
                              sems[MEGA % 2]).wait()
        transpose_and_scatter(bufs[MEGA % 2], 1, EPW)

    plsc.subcore_barrier()

    pltpu.sync_copy(sum_s.at[pl.ds(base_row, STRIPE)],
                    sums_hbm.at[c, pl.ds(base_row, STRIPE)])
    pltpu.sync_copy(cnt_s.at[pl.ds(base_row, STRIPE)],
                    counts_hbm.at[c, pl.ds(base_row, STRIPE)])

    @pl.when(s == 0)
    def _():
      pltpu.sync_copy(sum_s.at[pl.ds(NUM_SUBCORES * STRIPE, TAIL)],
                      sums_hbm.at[c, pl.ds(NUM_SUBCORES * STRIPE, TAIL)])
      pltpu.sync_copy(cnt_s.at[pl.ds(NUM_SUBCORES * STRIPE, TAIL)],
                      counts_hbm.at[c, pl.ds(NUM_SUBCORES * STRIPE, TAIL)])

  return k(dst, ea3)


def _tc_finish_body(x_ref, u_ref, s_ref, c_ref, o_ref):
  total = s_ref[0] + s_ref[1]
  cnt = c_ref[0] + c_ref[1]
  agg_t = (total / jnp.maximum(cnt, 1.0)).T
  x_t = x_ref[...].T
  u_b = jnp.broadcast_to(u_ref[...], (D_GLOBAL, N_NODES))
  o_ref[...] = jnp.concatenate([agg_t, x_t, u_b], axis=0)


def _tc_finish(x, u2d, sums, counts):
  out_t = pl.pallas_call(
      _tc_finish_body,
      out_shape=jax.ShapeDtypeStruct(
          (D_EDGE + D_FEAT + D_GLOBAL, N_NODES), jnp.float32),
  )(x, u2d, sums, counts)
  return out_t.T


@jax.jit
def kernel(x, edge_attr, u, edge_index):
  dst = edge_index.astype(jnp.int32)
  ea3 = edge_attr.T.reshape(D_EDGE, NGROUPS, G).transpose(1, 0, 2)
  sums, counts = _sc_segment_sum(dst, ea3)
  return _tc_finish(x, u.reshape(D_GLOBAL, 1), sums, counts)
